# baseline (speedup 1.0000x reference)
import functools

import jax
import jax.numpy as jnp
from jax import lax
from jax.experimental import pallas as pl
from jax.experimental.pallas import tpu as pltpu
from jax.experimental.pallas import tpu_sc as plsc

_ROWS_PER_TILE = 1024


def _argmin_body(z_ref, wt_ref, wsq_ref, zsq_ref, idx_ref, loss_ref, *,
                 n_steps, scale, k_half):
    i = pl.program_id(0)
    z = z_ref[...]
    wt = wt_ref[...]
    wsq = wsq_ref[...]
    zsq = zsq_ref[...]
    zb = (2.0 * z).astype(jnp.bfloat16).astype(jnp.float32)
    mm2 = jnp.dot(zb, wt, preferred_element_type=jnp.float32)
    scores = (zsq - mm2) + wsq
    lo = scores[:, :k_half]
    hi = scores[:, k_half:]
    m = z.shape[0]
    iota = lax.broadcasted_iota(jnp.int32, (m, k_half), 1).astype(jnp.float32)
    big = float(k_half)
    m_lo = jnp.min(lo, axis=1)
    a_lo = jnp.min(jnp.where(lo == m_lo[:, None], iota, big), axis=1).astype(jnp.int32)
    m_hi = jnp.min(hi, axis=1)
    a_hi = jnp.min(jnp.where(hi == m_hi[:, None], iota, big), axis=1).astype(jnp.int32)
    b_lo = m_lo.astype(jnp.bfloat16).astype(jnp.float32)
    choose_hi = m_hi < b_lo
    idx = jnp.where(choose_hi, a_hi + k_half, a_lo)
    mins = jnp.where(choose_hi, m_hi, m_lo)
    idx_ref[...] = idx.astype(jnp.int32)[:, None]
    part = jnp.sum(mins)

    @pl.when(i == 0)
    def _init():
        loss_ref[...] = jnp.zeros((1, 1), jnp.float32)

    loss_ref[...] += part.reshape(1, 1)

    @pl.when(i == n_steps - 1)
    def _finish():
        loss_ref[...] = loss_ref[...] * scale


def _tc_argmin(zf, wt, wsq, zsq, total_rows=None):
    n, d = zf.shape
    k = wt.shape[1]
    m = _ROWS_PER_TILE
    n_steps = n // m
    scale = 1.25 / float((total_rows or n) * d)
    idx, loss = pl.pallas_call(
        functools.partial(_argmin_body, n_steps=n_steps, scale=scale,
                          k_half=k // 2),
        grid=(n_steps,),
        in_specs=[
            pl.BlockSpec((m, d), lambda i: (i, 0)),
            pl.BlockSpec((d, k), lambda i: (0, 0)),
            pl.BlockSpec((1, k), lambda i: (0, 0)),
            pl.BlockSpec((m, 1), lambda i: (i, 0)),
        ],
        out_specs=[
            pl.BlockSpec((m, 1), lambda i: (i, 0)),
            pl.BlockSpec((1, 1), lambda i: (0, 0)),
        ],
        out_shape=[
            jax.ShapeDtypeStruct((n, 1), jnp.int32),
            jax.ShapeDtypeStruct((1, 1), jnp.float32),
        ],
    )(zf, wt, wsq, zsq)
    return idx.reshape(n), loss.reshape(())


def _make_sc_gather(n, k, d):
    info = plsc.get_sparse_core_info()
    nc, ns = info.num_cores, info.num_subcores
    nw = nc * ns
    assert n % (8 * nw) == 0 and d % info.num_lanes == 0
    b_per_w = n // nw
    mesh = plsc.VectorSubcoreMesh(core_axis_name="c", subcore_axis_name="s")

    @functools.partial(
        pl.kernel,
        mesh=mesh,
        compiler_params=pltpu.CompilerParams(use_tc_tiling_on_sc=False),
        out_type=jax.ShapeDtypeStruct((n, d), jnp.float32),
        scratch_types=[
            pltpu.VMEM((b_per_w,), jnp.int32),
            pltpu.VMEM((b_per_w, d), jnp.float32),
            pltpu.SemaphoreType.DMA,
        ],
    )
    def gather(table_hbm, idx_hbm, out_hbm, idx_v, rows_v, sem):
        wid = lax.axis_index("s") * nc + lax.axis_index("c")
        base = wid * b_per_w
        pltpu.sync_copy(idx_hbm.at[pl.ds(base, b_per_w)], idx_v)
        pltpu.async_copy(table_hbm.at[idx_v], rows_v, sem).wait()
        pltpu.sync_copy(rows_v, out_hbm.at[pl.ds(base, b_per_w)])

    return gather


def kernel(z, W):
    b, t, d = z.shape
    k = W.shape[0]
    n = b * t
    zf = z.reshape(n, d)
    wt = W.T
    wsq = jnp.sum(W**2, axis=1)[None, :]
    zsq = jnp.sum(z**2, axis=2).reshape(n, 1)
    n_chunks = 4
    rows = n // n_chunks
    gather = _make_sc_gather(rows, k, d)
    idx_parts, zq_parts, loss_parts = [], [], []
    for c in range(n_chunks):
        sl = slice(c * rows, (c + 1) * rows)
        idx_c, loss_c = _tc_argmin(zf[sl], wt, wsq, zsq[sl], total_rows=n)
        zq_parts.append(gather(W, idx_c))
        idx_parts.append(idx_c)
        loss_parts.append(loss_c)
    idx = jnp.concatenate(idx_parts)
    z_q = jnp.concatenate(zq_parts)
    loss = sum(loss_parts)
    return z_q.reshape(z.shape), loss, idx.reshape(b, t)


# speedup vs baseline: 1.1247x; 1.1247x over previous
import functools

import jax
import jax.numpy as jnp
from jax import lax
from jax.experimental import pallas as pl
from jax.experimental.pallas import tpu as pltpu
from jax.experimental.pallas import tpu_sc as plsc

_ROWS_PER_TILE = 1024


def _argmin_body(z_ref, wt_ref, wsq_ref, zsq_ref, idx_ref, loss_ref, *,
                 n_steps, scale, k_half):
    i = pl.program_id(0)
    z = z_ref[...]
    wt = wt_ref[...]
    wsq = wsq_ref[...]
    zsq = zsq_ref[...]
    zb = (2.0 * z).astype(jnp.bfloat16).astype(jnp.float32)
    mm2 = jnp.dot(zb, wt, preferred_element_type=jnp.float32)
    scores = (zsq - mm2) + wsq
    lo = scores[:, :k_half]
    hi = scores[:, k_half:]
    m = z.shape[0]
    iota = lax.broadcasted_iota(jnp.int32, (m, k_half), 1).astype(jnp.float32)
    big = float(k_half)
    m_lo = jnp.min(lo, axis=1)
    a_lo = jnp.min(jnp.where(lo == m_lo[:, None], iota, big), axis=1).astype(jnp.int32)
    m_hi = jnp.min(hi, axis=1)
    a_hi = jnp.min(jnp.where(hi == m_hi[:, None], iota, big), axis=1).astype(jnp.int32)
    b_lo = m_lo.astype(jnp.bfloat16).astype(jnp.float32)
    choose_hi = m_hi < b_lo
    idx = jnp.where(choose_hi, a_hi + k_half, a_lo)
    mins = jnp.where(choose_hi, m_hi, m_lo)
    idx_ref[...] = idx.astype(jnp.int32)[:, None]
    part = jnp.sum(mins)

    @pl.when(i == 0)
    def _init():
        loss_ref[...] = jnp.zeros((1, 1), jnp.float32)

    loss_ref[...] += part.reshape(1, 1)

    @pl.when(i == n_steps - 1)
    def _finish():
        loss_ref[...] = loss_ref[...] * scale


def _tc_argmin(zf, wt, wsq, zsq):
    n, d = zf.shape
    k = wt.shape[1]
    m = _ROWS_PER_TILE
    n_steps = n // m
    scale = 1.25 / float(n * d)
    idx, loss = pl.pallas_call(
        functools.partial(_argmin_body, n_steps=n_steps, scale=scale,
                          k_half=k // 2),
        grid=(n_steps,),
        in_specs=[
            pl.BlockSpec((m, d), lambda i: (i, 0)),
            pl.BlockSpec((d, k), lambda i: (0, 0)),
            pl.BlockSpec((1, k), lambda i: (0, 0)),
            pl.BlockSpec((m, 1), lambda i: (i, 0)),
        ],
        out_specs=[
            pl.BlockSpec((m, 1), lambda i: (i, 0)),
            pl.BlockSpec((1, 1), lambda i: (0, 0)),
        ],
        out_shape=[
            jax.ShapeDtypeStruct((n, 1), jnp.int32),
            jax.ShapeDtypeStruct((1, 1), jnp.float32),
        ],
    )(zf, wt, wsq, zsq)
    return idx.reshape(n), loss.reshape(())


def _make_sc_gather(n, k, d):
    info = plsc.get_sparse_core_info()
    nc, ns = info.num_cores, info.num_subcores
    nw = nc * ns
    assert n % (8 * nw) == 0 and d % info.num_lanes == 0
    b_per_w = n // nw
    mesh = plsc.VectorSubcoreMesh(core_axis_name="c", subcore_axis_name="s")

    @functools.partial(
        pl.kernel,
        mesh=mesh,
        compiler_params=pltpu.CompilerParams(use_tc_tiling_on_sc=False),
        out_type=jax.ShapeDtypeStruct((n, d), jnp.float32),
        scratch_types=[
            pltpu.VMEM((b_per_w,), jnp.int32),
            pltpu.VMEM((b_per_w, d), jnp.float32),
            pltpu.SemaphoreType.DMA,
        ],
    )
    def gather(table_hbm, idx_hbm, out_hbm, idx_v, rows_v, sem):
        wid = lax.axis_index("s") * nc + lax.axis_index("c")
        base = wid * b_per_w
        pltpu.sync_copy(idx_hbm.at[pl.ds(base, b_per_w)], idx_v)
        pltpu.async_copy(table_hbm.at[idx_v], rows_v, sem).wait()
        pltpu.sync_copy(rows_v, out_hbm.at[pl.ds(base, b_per_w)])

    return gather


def kernel(z, W):
    b, t, d = z.shape
    k = W.shape[0]
    n = b * t
    zf = z.reshape(n, d)
    wsq = jnp.sum(W**2, axis=1)[None, :]
    zsq = jnp.sum(z**2, axis=2).reshape(n, 1)
    idx, loss = _tc_argmin(zf, W.T, wsq, zsq)
    z_q = _make_sc_gather(n, k, d)(W, idx)
    return z_q.reshape(z.shape), loss, idx.reshape(b, t)
